# baseline (speedup 1.0000x reference)
import functools
import math

import jax
import jax.numpy as jnp
from jax.experimental import pallas as pl
from jax.experimental.pallas import tpu as pltpu

E = 8
K = 2
HID = 1024
INTER = 2048
NT = 4
TILE_I = INTER // NT


def _ffn_body(x_ref, w1_ref, w2_ref, y_ref, acc_ref):
    nt = pl.program_id(1)
    h = jnp.dot(x_ref[0], w1_ref[0], preferred_element_type=jnp.float32)
    h = jnp.maximum(h, 0.0)
    part = jnp.dot(h, w2_ref[0], preferred_element_type=jnp.float32)

    @pl.when(nt == 0)
    def _():
        acc_ref[...] = part

    @pl.when(nt > 0)
    def _():
        acc_ref[...] = acc_ref[...] + part

    @pl.when(nt == NT - 1)
    def _():
        y_ref[0] = acc_ref[...]


def _ffn(x_disp, experts_inter, experts_out, cap):
    return pl.pallas_call(
        _ffn_body,
        grid=(E, NT),
        in_specs=[
            pl.BlockSpec((1, cap, HID), lambda e, n: (e, 0, 0)),
            pl.BlockSpec((1, HID, TILE_I), lambda e, n: (e, 0, n)),
            pl.BlockSpec((1, TILE_I, HID), lambda e, n: (e, n, 0)),
        ],
        out_specs=pl.BlockSpec((1, cap, HID), lambda e, n: (e, 0, 0)),
        out_shape=jax.ShapeDtypeStruct((E, cap, HID), jnp.float32),
        scratch_shapes=[pltpu.VMEM((cap, HID), jnp.float32)],
        compiler_params=pltpu.CompilerParams(
            dimension_semantics=("arbitrary", "arbitrary"),
        ),
    )(x_disp, experts_inter, experts_out)


def kernel(x, experts_inter, experts_out, router_w, router_b):
    b, s, hid = x.shape
    T = b * s
    cap = math.ceil(T / E * 1.0)
    x_flat = x.reshape(T, hid)

    logits = x_flat @ router_w.T + router_b
    probs = jax.nn.softmax(logits, axis=-1)
    rows = jnp.arange(T)
    i1 = jnp.argmax(probs, axis=-1)
    v1 = jnp.take_along_axis(probs, i1[:, None], axis=-1)[:, 0]
    masked = probs.at[rows, i1].set(-jnp.inf)
    i2 = jnp.argmax(masked, axis=-1)
    v2 = jnp.take_along_axis(probs, i2[:, None], axis=-1)[:, 0]

    mask = jnp.zeros((T, E), jnp.int32).at[rows, i1].set(1)
    mask = mask.at[rows, i2].set(1)
    slots = jnp.cumsum(mask, axis=0) - mask
    c1 = jnp.take_along_axis(slots, i1[:, None], axis=-1)[:, 0]
    c2 = jnp.take_along_axis(slots, i2[:, None], axis=-1)[:, 0]
    ok1 = c1 < cap
    ok2 = c2 < cap
    w1 = jnp.where(ok1, v1, 0.0)
    w2 = jnp.where(ok2, v2, 0.0)
    flat1 = jnp.where(ok1, i1 * cap + c1, 0)
    flat2 = jnp.where(ok2, i2 * cap + c2, 0)

    X = jnp.zeros((E * cap, hid), x.dtype)
    X = X.at[flat1].add(jnp.where(ok1[:, None], x_flat, 0.0))
    X = X.at[flat2].add(jnp.where(ok2[:, None], x_flat, 0.0))

    Y = _ffn(X.reshape(E, cap, hid), experts_inter, experts_out, cap)
    Y = Y.reshape(E * cap, hid)

    out = w1[:, None] * Y[flat1] + w2[:, None] * Y[flat2]
    return out.reshape(b, s, hid)
